# baseline (speedup 1.0000x reference)
import jax
import jax.numpy as jnp
from jax import lax
from jax.experimental import pallas as pl
from jax.experimental.pallas import tpu as pltpu
from jax.experimental.pallas import tpu_sc as plsc

N = 10000
EG = 320000
IN_DIM = 128
D = 16
H = 8
HD = H * D
CLAMP = 5.0

NC = 2
NS = 16
NW = NC * NS

CHUNK = 80
GRP = 4
GEDGES = GRP * CHUNK
SGRP = 2
SEDGES = SGRP * CHUNK

H1E = 163840
H2E = EG - H1E

ROWS_PT = 640
NPAD = ROWS_PT * NS
LAST_ROWS = N - ROWS_PT * (NS - 1)

KB = 1280


def _signed_sqrt(v):
    safe_p = jnp.where(v > 0, v, 1.0)
    safe_n = jnp.where(v < 0, -v, 1.0)
    pos = jnp.where(v > 0, jnp.sqrt(safe_p), 0.0)
    neg = jnp.where(v < 0, jnp.sqrt(safe_n), 0.0)
    return pos - neg


def _k1_body(x_ref, w_ref, b_ref, q_ref, kv_ref):
    tab = jnp.dot(x_ref[...], w_ref[...].T,
                  preferred_element_type=jnp.float32) + b_ref[...]
    q_ref[...] = tab[:, 0:HD]
    kv_ref[...] = tab[:, HD:3 * HD].astype(jnp.bfloat16)


def _project(x, wall, ball):
    return pl.pallas_call(
        _k1_body,
        out_shape=(jax.ShapeDtypeStruct((N, HD), jnp.float32),
                   jax.ShapeDtypeStruct((N, 2 * HD), jnp.bfloat16)),
    )(x, wall, ball)


def _gather_edges(kvtab, qtab, src, dst, e0, ne):
    cpw = ne // NW // CHUNK
    ngrp = cpw // GRP
    leftover = cpw - ngrp * GRP

    def body(kvtab, qtab, src_hbm, dst_hbm, kvg, qg,
             sidx, didx, kvbuf, qbuf, isem, wk, wq):
        cid = lax.axis_index("c")
        sid = lax.axis_index("s")
        wid = sid * NC + cid
        crbase = wid * cpw

        def fire_gathers(nch):
            descs = []
            for j in range(nch):
                descs.append(pltpu.async_copy(
                    kvtab.at[sidx.at[pl.ds(j * CHUNK, CHUNK)]],
                    kvbuf.at[pl.ds(j * CHUNK, CHUNK)], isem))
                descs.append(pltpu.async_copy(
                    qtab.at[didx.at[pl.ds(j * CHUNK, CHUNK)]],
                    qbuf.at[pl.ds(j * CHUNK, CHUNK)], isem))
            for d in descs:
                d.wait()

        def step(t, carry):
            lbase = (crbase + t * GRP) * CHUNK

            @pl.when(t > 0)
            def _():
                pltpu.make_async_copy(kvbuf, kvg.at[pl.ds(lbase, GEDGES)],
                                      wk).wait()
                pltpu.make_async_copy(qbuf, qg.at[pl.ds(lbase, GEDGES)],
                                      wq).wait()

            pltpu.sync_copy(src_hbm.at[pl.ds(e0 + lbase, GEDGES)], sidx)
            pltpu.sync_copy(dst_hbm.at[pl.ds(e0 + lbase, GEDGES)], didx)
            fire_gathers(GRP)
            pltpu.async_copy(kvbuf, kvg.at[pl.ds(lbase, GEDGES)], wk)
            pltpu.async_copy(qbuf, qg.at[pl.ds(lbase, GEDGES)], wq)
            return carry

        lax.fori_loop(0, ngrp, step, 0)

        lbase = (crbase + ngrp * GRP) * CHUNK
        pltpu.make_async_copy(kvbuf, kvg.at[pl.ds(lbase, GEDGES)], wk).wait()
        pltpu.make_async_copy(qbuf, qg.at[pl.ds(lbase, GEDGES)], wq).wait()
        if leftover:
            pltpu.sync_copy(src_hbm.at[pl.ds(e0 + lbase, CHUNK)],
                            sidx.at[pl.ds(0, CHUNK)])
            pltpu.sync_copy(dst_hbm.at[pl.ds(e0 + lbase, CHUNK)],
                            didx.at[pl.ds(0, CHUNK)])
            fire_gathers(1)
            pltpu.sync_copy(kvbuf.at[pl.ds(0, CHUNK)],
                            kvg.at[pl.ds(lbase, CHUNK)])
            pltpu.sync_copy(qbuf.at[pl.ds(0, CHUNK)],
                            qg.at[pl.ds(lbase, CHUNK)])

    mesh = plsc.VectorSubcoreMesh(core_axis_name="c", subcore_axis_name="s",
                                  num_cores=NC, num_subcores=NS)
    kern = pl.kernel(
        body,
        out_type=(jax.ShapeDtypeStruct((ne, HD), jnp.int32),
                  jax.ShapeDtypeStruct((ne, HD), jnp.float32)),
        mesh=mesh,
        scratch_types=[
            pltpu.VMEM((GEDGES,), jnp.int32),
            pltpu.VMEM((GEDGES,), jnp.int32),
            pltpu.VMEM((GEDGES, HD), jnp.int32),
            pltpu.VMEM((GEDGES, HD), jnp.float32),
            pltpu.SemaphoreType.DMA,
            pltpu.SemaphoreType.DMA,
            pltpu.SemaphoreType.DMA,
        ],
    )
    return kern(kvtab, qtab, src, dst)


def _k3_body(ea_ref, kvg_ref, qg_ref, eww_ref, ebw_ref,
             awb_ref, we_ref, ue_ref, uv_ref, pexp_ref):
    ea = ea_ref[...]
    eproj = jnp.dot(ea, eww_ref[...].T,
                    preferred_element_type=jnp.float32) + ebw_ref[...]
    ewp = eproj[:, 0:HD]
    ebp = eproj[:, HD:2 * HD]
    kvi = kvg_ref[...]
    kg = jax.lax.bitcast_convert_type(kvi << 16, jnp.float32)
    vg = jax.lax.bitcast_convert_type(kvi & jnp.int32(-65536), jnp.float32)
    g = kg + qg_ref[...]
    score = _signed_sqrt(g * ewp) + ebp
    we_ref[...] = score
    sb = jnp.dot(score, awb_ref[...], preferred_element_type=jnp.float32)
    pexp = jnp.exp(jnp.clip(sb, -CLAMP, CLAMP))
    pexp_ref[...] = pexp
    ue_ref[...] = score * pexp
    uv_ref[...] = vg * pexp


def _edge_compute(edge_attr, kvg, qg, eww, ebw, awb, e0, ne, we_carry):
    grid = ne // KB
    b0 = e0 // KB
    ea_spec = pl.BlockSpec((KB, HD), lambda i: (i + b0, 0))
    eg_spec = pl.BlockSpec((KB, HD), lambda i: (i, 0))
    full = lambda r, c: pl.BlockSpec((r, c), lambda i: (0, 0))
    eg_out = jax.ShapeDtypeStruct((ne, HD), jnp.float32)
    in_specs = [ea_spec, eg_spec, eg_spec,
                full(2 * HD, IN_DIM), full(1, 2 * HD), full(HD, HD)]
    args = [edge_attr, kvg, qg, eww, ebw, awb]
    aliases = {}
    body = _k3_body
    if we_carry is not None:
        in_specs.append(pl.BlockSpec(memory_space=pl.ANY))
        args.append(we_carry)
        aliases = {6: 0}
        body = lambda ea, kvg_, qg_, eww_, ebw_, awb_, wein, *outs: \
            _k3_body(ea, kvg_, qg_, eww_, ebw_, awb_, *outs)
    return pl.pallas_call(
        body,
        grid=(grid,),
        in_specs=in_specs,
        out_specs=[ea_spec, eg_spec, eg_spec, eg_spec],
        out_shape=[jax.ShapeDtypeStruct((EG, HD), jnp.float32),
                   eg_out, eg_out, eg_out],
        input_output_aliases=aliases,
    )(*args)


def _scatter_edges(uv, ue, pexp, dst, z128, e0, ne):
    SCH = 128
    pt1 = ne // NS
    pt2 = ne // 2 // NS
    n1, t1 = pt1 // SCH, pt1 % SCH
    n2, t2 = pt2 // SCH, pt2 % SCH
    tails = sorted({t for t in (t1, t2) if t})

    def body(uv_hbm, ue_hbm, pexp_hbm, dst_hbm, z128,
             accv_out, acce_out, ssa_out, ssb_out,
             sp128, idxa, idxb, bufa, bufb, *rest):
        tail_idx = {t: rest[i] for i, t in enumerate(tails)}
        isem, ssa_sem, ssb_sem = rest[len(tails):]
        cid = lax.axis_index("c")
        sid = lax.axis_index("s")
        rbase = sid * ROWS_PT

        def _zero():
            pltpu.sync_copy(z128.at[pl.ds(rbase, ROWS_PT)],
                            sp128.at[pl.ds(rbase, ROWS_PT)])

        def _load(val_hbm, lbase, ixb, dbuf, nr=SCH):
            d1 = pltpu.async_copy(
                dst_hbm.at[pl.ds(e0 + lbase, nr)], ixb, isem)
            d2 = pltpu.async_copy(
                val_hbm.at[pl.ds(lbase, nr)], dbuf.at[pl.ds(0, nr)], isem)
            d1.wait()
            d2.wait()

        def _fire(ixb, dbuf, sem, nr=SCH):
            pltpu.async_copy(dbuf.at[pl.ds(0, nr)],
                             sp128.at[ixb], sem, add=True)

        def _drain(ixb, dbuf, sem, nr=SCH):
            pltpu.make_async_copy(dbuf.at[pl.ds(0, nr)],
                                  sp128.at[ixb], sem).wait()

        def _scan(val_hbm, lbase0, n_chunks, tail):
            n_pairs = n_chunks // 2
            rem = n_chunks - 2 * n_pairs

            def step(t, carry):
                base_a = lbase0 + (2 * t) * SCH

                @pl.when(t > 0)
                def _():
                    _drain(idxa, bufa, ssa_sem)

                _load(val_hbm, base_a, idxa, bufa)
                _fire(idxa, bufa, ssa_sem)

                @pl.when(t > 0)
                def _():
                    _drain(idxb, bufb, ssb_sem)

                _load(val_hbm, base_a + SCH, idxb, bufb)
                _fire(idxb, bufb, ssb_sem)
                return carry
            lax.fori_loop(0, n_pairs, step, 0)
            _drain(idxa, bufa, ssa_sem)
            base = lbase0 + 2 * n_pairs * SCH
            if rem:
                _load(val_hbm, base, idxa, bufa)
                _fire(idxa, bufa, ssa_sem)
                _drain(idxa, bufa, ssa_sem)
                base += SCH
            if tail:
                ixt = tail_idx[tail]
                _load(val_hbm, base, ixt, bufa, tail)
                _fire(ixt, bufa, ssa_sem, tail)
                _drain(ixt, bufa, ssa_sem, tail)
            _drain(idxb, bufb, ssb_sem)

        def _writeback(out_hbm):
            @pl.when(sid < NS - 1)
            def _():
                pltpu.sync_copy(sp128.at[pl.ds(rbase, ROWS_PT)],
                                out_hbm.at[pl.ds(rbase, ROWS_PT)])

            @pl.when(sid == NS - 1)
            def _():
                pltpu.sync_copy(sp128.at[pl.ds(rbase, LAST_ROWS)],
                                out_hbm.at[pl.ds(rbase, LAST_ROWS)])

        _zero()
        plsc.subcore_barrier()

        @pl.when(cid == 0)
        def _():
            _scan(uv_hbm, sid * pt1, n1, t1)

        @pl.when(cid == 1)
        def _():
            _scan(ue_hbm, sid * pt1, n1, t1)

        plsc.subcore_barrier()

        @pl.when(cid == 0)
        def _():
            _writeback(accv_out)

        @pl.when(cid == 1)
        def _():
            _writeback(acce_out)

        plsc.subcore_barrier()

        _zero()
        plsc.subcore_barrier()
        _scan(pexp_hbm, cid * (ne // 2) + sid * pt2, n2, t2)
        plsc.subcore_barrier()

        @pl.when(cid == 0)
        def _():
            _writeback(ssa_out)

        @pl.when(cid == 1)
        def _():
            _writeback(ssb_out)

    mesh = plsc.VectorSubcoreMesh(core_axis_name="c", subcore_axis_name="s",
                                  num_cores=NC, num_subcores=NS)
    n_out = jax.ShapeDtypeStruct((N, HD), jnp.float32)
    kern = pl.kernel(
        body,
        out_type=(n_out, n_out, n_out, n_out),
        mesh=mesh,
        scratch_types=[
            pltpu.VMEM_SHARED((NPAD, HD), jnp.float32),
            pltpu.VMEM((SCH,), jnp.int32),
            pltpu.VMEM((SCH,), jnp.int32),
            pltpu.VMEM((SCH, HD), jnp.float32),
            pltpu.VMEM((SCH, HD), jnp.float32),
        ] + [pltpu.VMEM((t,), jnp.int32) for t in tails] + [
            pltpu.SemaphoreType.DMA,
            pltpu.SemaphoreType.DMA,
            pltpu.SemaphoreType.DMA,
        ],
    )
    return kern(uv, ue, pexp, dst, z128)


def _k5_body(av1, av2, ae1, ae2, sa1, sb1, sa2, sb2, vrp_ref, out_ref):
    ssum = sa1[...] + sb1[...] + sa2[...] + sb2[...]
    rexp = 1.0 / (ssum + 1e-16)
    acce = ae1[...] + ae2[...]
    rowv = jnp.dot(acce * rexp, vrp_ref[...],
                   preferred_element_type=jnp.float32)
    out_ref[...] = (av1[...] + av2[...]) * rexp + rowv


def _finalize(accs1, accs2, vrp):
    av1, ae1, sa1, sb1 = accs1
    av2, ae2, sa2, sb2 = accs2
    return pl.pallas_call(
        _k5_body,
        out_shape=jax.ShapeDtypeStruct((N, HD), jnp.float32),
    )(av1, av2, ae1, ae2, sa1, sb1, sa2, sb2, vrp)


def kernel(x, edge_attr, edge_index, Qw, Qb, Kw, Kb, Ew, Eb, Vw, Vb, Aw, VeRow):
    f32 = jnp.float32
    x = x.astype(f32)
    edge_attr = edge_attr.astype(f32)
    src = edge_index[0].astype(jnp.int32)
    dst = edge_index[1].astype(jnp.int32)

    kvw = jnp.stack([Kw, Vw], axis=1).reshape(2 * HD, IN_DIM)
    kvb = jnp.stack([Kb, Vb], axis=1).reshape(2 * HD)
    wall = jnp.concatenate([Qw, kvw], axis=0)
    ball = jnp.concatenate([Qb, kvb])[None, :]

    ew3 = Ew.reshape(H, 2 * D, IN_DIM)
    eww = jnp.concatenate([ew3[:, :D].reshape(HD, IN_DIM),
                           ew3[:, D:].reshape(HD, IN_DIM)], axis=0)
    eb3 = Eb.reshape(H, 2 * D)
    ebw = jnp.concatenate([eb3[:, :D].reshape(HD),
                           eb3[:, D:].reshape(HD)])[None, :]

    eye_h = jnp.eye(H, dtype=f32)
    aw8 = (Aw[:, :, 0].T[:, :, None] * eye_h[:, None, :]).reshape(HD, H)
    awb = jnp.repeat(aw8, D, axis=1)
    vrp = (VeRow.transpose(1, 0, 2)[:, :, None, :] *
           eye_h[:, None, :, None]).reshape(HD, HD)

    z128 = jnp.zeros((NPAD, HD), f32)

    qtab, kv16 = _project(x, wall, ball)
    kvtab = jax.lax.bitcast_convert_type(
        kv16.reshape(N, HD, 2), jnp.int32)

    halves = ((0, H1E), (H1E, H2E))
    gathered = [_gather_edges(kvtab, qtab, src, dst, e0, ne)
                for (e0, ne) in halves]
    batch_wE, accs = None, []
    for (e0, ne), (kvg, qg) in zip(halves, gathered):
        batch_wE, ue_h, uv_h, pexp_h = _edge_compute(
            edge_attr, kvg, qg, eww, ebw, awb, e0, ne, batch_wE)
        accs.append(_scatter_edges(uv_h, ue_h, pexp_h, dst, z128, e0, ne))

    batch_wV = _finalize(accs[0], accs[1], vrp)
    return batch_wV.reshape(N, H, D), batch_wE


# speedup vs baseline: 1.0214x; 1.0214x over previous
import jax
import jax.numpy as jnp
from jax import lax
from jax.experimental import pallas as pl
from jax.experimental.pallas import tpu as pltpu
from jax.experimental.pallas import tpu_sc as plsc

N = 10000
EG = 320000
IN_DIM = 128
D = 16
H = 8
HD = H * D
CLAMP = 5.0

NC = 2
NS = 16
NW = NC * NS

CHUNK = 80
GRP = 4
GEDGES = GRP * CHUNK
SGRP = 2
SEDGES = SGRP * CHUNK

H1E = 163840
H2E = EG - H1E

ROWS_PT = 640
NPAD = ROWS_PT * NS
LAST_ROWS = N - ROWS_PT * (NS - 1)

KB = 1280


def _signed_sqrt(v):
    safe_p = jnp.where(v > 0, v, 1.0)
    safe_n = jnp.where(v < 0, -v, 1.0)
    pos = jnp.where(v > 0, jnp.sqrt(safe_p), 0.0)
    neg = jnp.where(v < 0, jnp.sqrt(safe_n), 0.0)
    return pos - neg


def _k1_body(x_ref, w_ref, b_ref, q_ref, kv_ref):
    tab = jnp.dot(x_ref[...], w_ref[...].T,
                  preferred_element_type=jnp.float32) + b_ref[...]
    q_ref[...] = tab[:, 0:HD]
    kv_ref[...] = tab[:, HD:3 * HD].astype(jnp.bfloat16)


def _project(x, wall, ball):
    return pl.pallas_call(
        _k1_body,
        out_shape=(jax.ShapeDtypeStruct((N, HD), jnp.float32),
                   jax.ShapeDtypeStruct((N, 2 * HD), jnp.bfloat16)),
    )(x, wall, ball)


def _gather_edges(kvtab, qtab, src, dst, e0, ne):
    cpw = ne // NW // CHUNK
    ngrp = cpw // GRP
    leftover = cpw - ngrp * GRP

    def body(kvtab, qtab, src_hbm, dst_hbm, kvg, qg,
             sidx, didx, kvbuf, qbuf, isem, wk, wq):
        cid = lax.axis_index("c")
        sid = lax.axis_index("s")
        wid = sid * NC + cid
        crbase = wid * cpw

        def fire_gathers(nch):
            descs = []
            for j in range(nch):
                descs.append(pltpu.async_copy(
                    kvtab.at[sidx.at[pl.ds(j * CHUNK, CHUNK)]],
                    kvbuf.at[pl.ds(j * CHUNK, CHUNK)], isem))
                descs.append(pltpu.async_copy(
                    qtab.at[didx.at[pl.ds(j * CHUNK, CHUNK)]],
                    qbuf.at[pl.ds(j * CHUNK, CHUNK)], isem))
            for d in descs:
                d.wait()

        def step(t, carry):
            lbase = (crbase + t * GRP) * CHUNK

            @pl.when(t > 0)
            def _():
                pltpu.make_async_copy(kvbuf, kvg.at[pl.ds(lbase, GEDGES)],
                                      wk).wait()
                pltpu.make_async_copy(qbuf, qg.at[pl.ds(lbase, GEDGES)],
                                      wq).wait()

            pltpu.sync_copy(src_hbm.at[pl.ds(e0 + lbase, GEDGES)], sidx)
            pltpu.sync_copy(dst_hbm.at[pl.ds(e0 + lbase, GEDGES)], didx)
            fire_gathers(GRP)
            pltpu.async_copy(kvbuf, kvg.at[pl.ds(lbase, GEDGES)], wk)
            pltpu.async_copy(qbuf, qg.at[pl.ds(lbase, GEDGES)], wq)
            return carry

        lax.fori_loop(0, ngrp, step, 0)

        lbase = (crbase + ngrp * GRP) * CHUNK
        pltpu.make_async_copy(kvbuf, kvg.at[pl.ds(lbase, GEDGES)], wk).wait()
        pltpu.make_async_copy(qbuf, qg.at[pl.ds(lbase, GEDGES)], wq).wait()
        if leftover:
            pltpu.sync_copy(src_hbm.at[pl.ds(e0 + lbase, CHUNK)],
                            sidx.at[pl.ds(0, CHUNK)])
            pltpu.sync_copy(dst_hbm.at[pl.ds(e0 + lbase, CHUNK)],
                            didx.at[pl.ds(0, CHUNK)])
            fire_gathers(1)
            pltpu.sync_copy(kvbuf.at[pl.ds(0, CHUNK)],
                            kvg.at[pl.ds(lbase, CHUNK)])
            pltpu.sync_copy(qbuf.at[pl.ds(0, CHUNK)],
                            qg.at[pl.ds(lbase, CHUNK)])

    mesh = plsc.VectorSubcoreMesh(core_axis_name="c", subcore_axis_name="s",
                                  num_cores=NC, num_subcores=NS)
    kern = pl.kernel(
        body,
        out_type=(jax.ShapeDtypeStruct((ne, HD), jnp.int32),
                  jax.ShapeDtypeStruct((ne, HD), jnp.float32)),
        mesh=mesh,
        scratch_types=[
            pltpu.VMEM((GEDGES,), jnp.int32),
            pltpu.VMEM((GEDGES,), jnp.int32),
            pltpu.VMEM((GEDGES, HD), jnp.int32),
            pltpu.VMEM((GEDGES, HD), jnp.float32),
            pltpu.SemaphoreType.DMA,
            pltpu.SemaphoreType.DMA,
            pltpu.SemaphoreType.DMA,
        ],
    )
    return kern(kvtab, qtab, src, dst)


def _k3_body(ea_ref, kvg_ref, qg_ref, eww_ref, ebw_ref,
             awb_ref, we_ref, ue_ref, uv_ref, pexp_ref):
    ea = ea_ref[...]
    eproj = jnp.dot(ea, eww_ref[...].T,
                    preferred_element_type=jnp.float32) + ebw_ref[...]
    ewp = eproj[:, 0:HD]
    ebp = eproj[:, HD:2 * HD]
    kvi = kvg_ref[...]
    kg = jax.lax.bitcast_convert_type(kvi << 16, jnp.float32)
    vg = jax.lax.bitcast_convert_type(kvi & jnp.int32(-65536), jnp.float32)
    g = kg + qg_ref[...]
    score = _signed_sqrt(g * ewp) + ebp
    we_ref[...] = score
    sb = jnp.dot(score, awb_ref[...], preferred_element_type=jnp.float32)
    pexp = jnp.exp(jnp.clip(sb, -CLAMP, CLAMP))
    pexp_ref[...] = pexp
    ue_ref[...] = score * pexp
    uv_ref[...] = vg * pexp


def _edge_compute(edge_attr, kvg, qg, eww, ebw, awb, e0, ne, we_carry):
    grid = ne // KB
    b0 = e0 // KB
    ea_spec = pl.BlockSpec((KB, HD), lambda i: (i + b0, 0))
    eg_spec = pl.BlockSpec((KB, HD), lambda i: (i, 0))
    full = lambda r, c: pl.BlockSpec((r, c), lambda i: (0, 0))
    eg_out = jax.ShapeDtypeStruct((ne, HD), jnp.float32)
    in_specs = [ea_spec, eg_spec, eg_spec,
                full(2 * HD, IN_DIM), full(1, 2 * HD), full(HD, HD)]
    args = [edge_attr, kvg, qg, eww, ebw, awb]
    aliases = {}
    body = _k3_body
    if we_carry is not None:
        in_specs.append(pl.BlockSpec(memory_space=pl.ANY))
        args.append(we_carry)
        aliases = {6: 0}
        body = lambda ea, kvg_, qg_, eww_, ebw_, awb_, wein, *outs: \
            _k3_body(ea, kvg_, qg_, eww_, ebw_, awb_, *outs)
    return pl.pallas_call(
        body,
        grid=(grid,),
        in_specs=in_specs,
        out_specs=[ea_spec, eg_spec, eg_spec, eg_spec],
        out_shape=[jax.ShapeDtypeStruct((EG, HD), jnp.float32),
                   eg_out, eg_out, eg_out],
        input_output_aliases=aliases,
    )(*args)


def _scatter_edges(uv, ue, pexp, dst, z128, e0, ne):
    n1 = (ne // NS) // CHUNK
    n2 = (ne // 2 // NS) // CHUNK
    pt1 = ne // NS
    pt2 = ne // 2 // NS
    t1 = 0
    t2 = 0

    def body(uv_hbm, ue_hbm, pexp_hbm, dst_hbm, z128,
             accv_out, acce_out, ssa_out, ssb_out,
             sp128, idxa, idxb, bufa, bufb, isem, ssa_sem, ssb_sem):
        cid = lax.axis_index("c")
        sid = lax.axis_index("s")
        rbase = sid * ROWS_PT

        def _zero():
            pltpu.sync_copy(z128.at[pl.ds(rbase, ROWS_PT)],
                            sp128.at[pl.ds(rbase, ROWS_PT)])

        def _load_group(val_hbm, lbase, ixb, dbuf, nch=SGRP):
            descs = []
            for j in range(nch):
                descs.append(pltpu.async_copy(
                    dst_hbm.at[pl.ds(e0 + lbase + j * CHUNK, CHUNK)],
                    ixb.at[j], isem))
            descs.append(pltpu.async_copy(
                val_hbm.at[pl.ds(lbase, nch * CHUNK)],
                dbuf.at[pl.ds(0, nch * CHUNK)], isem))
            for d in descs:
                d.wait()

        def _fire_scatter(ixb, dbuf, sem, nch=SGRP):
            for j in range(nch):
                pltpu.async_copy(dbuf.at[pl.ds(j * CHUNK, CHUNK)],
                                 sp128.at[ixb.at[j]], sem, add=True)

        def _drain_scatter(ixb, dbuf, sem, nch=SGRP):
            for j in range(nch):
                pltpu.make_async_copy(dbuf.at[pl.ds(j * CHUNK, CHUNK)],
                                      sp128.at[ixb.at[j]], sem).wait()

        def _scan(val_hbm, lbase0, n_chunks, tail=0):
            n_pairs = n_chunks // (2 * SGRP)
            rem = n_chunks - 2 * n_pairs * SGRP

            def step(t, carry):
                base_a = lbase0 + (2 * t) * SEDGES
                base_b = base_a + SEDGES

                @pl.when(t > 0)
                def _():
                    _drain_scatter(idxa, bufa, ssa_sem)

                _load_group(val_hbm, base_a, idxa, bufa)
                _fire_scatter(idxa, bufa, ssa_sem)

                @pl.when(t > 0)
                def _():
                    _drain_scatter(idxb, bufb, ssb_sem)

                _load_group(val_hbm, base_b, idxb, bufb)
                _fire_scatter(idxb, bufb, ssb_sem)
                return carry
            lax.fori_loop(0, n_pairs, step, 0)
            _drain_scatter(idxa, bufa, ssa_sem)
            base = lbase0 + 2 * n_pairs * SEDGES
            while rem > 0:
                nch = min(SGRP, rem)
                _load_group(val_hbm, base, idxa, bufa, nch)
                _fire_scatter(idxa, bufa, ssa_sem, nch)
                _drain_scatter(idxa, bufa, ssa_sem, nch)
                base += nch * CHUNK
                rem -= nch
            _drain_scatter(idxb, bufb, ssb_sem)

        def _writeback(out_hbm):
            @pl.when(sid < NS - 1)
            def _():
                pltpu.sync_copy(sp128.at[pl.ds(rbase, ROWS_PT)],
                                out_hbm.at[pl.ds(rbase, ROWS_PT)])

            @pl.when(sid == NS - 1)
            def _():
                pltpu.sync_copy(sp128.at[pl.ds(rbase, LAST_ROWS)],
                                out_hbm.at[pl.ds(rbase, LAST_ROWS)])

        _zero()
        plsc.subcore_barrier()

        @pl.when(cid == 0)
        def _():
            _scan(uv_hbm, sid * pt1, n1, t1)

        @pl.when(cid == 1)
        def _():
            _scan(ue_hbm, sid * pt1, n1, t1)

        plsc.subcore_barrier()

        @pl.when(cid == 0)
        def _():
            _writeback(accv_out)

        @pl.when(cid == 1)
        def _():
            _writeback(acce_out)

        plsc.subcore_barrier()

        _zero()
        plsc.subcore_barrier()
        _scan(pexp_hbm, cid * (ne // 2) + sid * pt2, n2, t2)
        plsc.subcore_barrier()

        @pl.when(cid == 0)
        def _():
            _writeback(ssa_out)

        @pl.when(cid == 1)
        def _():
            _writeback(ssb_out)

    mesh = plsc.VectorSubcoreMesh(core_axis_name="c", subcore_axis_name="s",
                                  num_cores=NC, num_subcores=NS)
    n_out = jax.ShapeDtypeStruct((N, HD), jnp.float32)
    kern = pl.kernel(
        body,
        out_type=(n_out, n_out, n_out, n_out),
        mesh=mesh,
        scratch_types=[
            pltpu.VMEM_SHARED((NPAD, HD), jnp.float32),
            pltpu.VMEM((SGRP, CHUNK), jnp.int32),
            pltpu.VMEM((SGRP, CHUNK), jnp.int32),
            pltpu.VMEM((SEDGES, HD), jnp.float32),
            pltpu.VMEM((SEDGES, HD), jnp.float32),
            pltpu.SemaphoreType.DMA,
            pltpu.SemaphoreType.DMA,
            pltpu.SemaphoreType.DMA,
        ],
    )
    return kern(uv, ue, pexp, dst, z128)


def _k5_body(av1, av2, ae1, ae2, sa1, sb1, sa2, sb2, vrp_ref, out_ref):
    ssum = sa1[...] + sb1[...] + sa2[...] + sb2[...]
    rexp = 1.0 / (ssum + 1e-16)
    acce = ae1[...] + ae2[...]
    rowv = jnp.dot(acce * rexp, vrp_ref[...],
                   preferred_element_type=jnp.float32)
    out_ref[...] = (av1[...] + av2[...]) * rexp + rowv


def _finalize(accs1, accs2, vrp):
    av1, ae1, sa1, sb1 = accs1
    av2, ae2, sa2, sb2 = accs2
    return pl.pallas_call(
        _k5_body,
        out_shape=jax.ShapeDtypeStruct((N, HD), jnp.float32),
    )(av1, av2, ae1, ae2, sa1, sb1, sa2, sb2, vrp)


def kernel(x, edge_attr, edge_index, Qw, Qb, Kw, Kb, Ew, Eb, Vw, Vb, Aw, VeRow):
    f32 = jnp.float32
    x = x.astype(f32)
    edge_attr = edge_attr.astype(f32)
    src = edge_index[0].astype(jnp.int32)
    dst = edge_index[1].astype(jnp.int32)

    kvw = jnp.stack([Kw, Vw], axis=1).reshape(2 * HD, IN_DIM)
    kvb = jnp.stack([Kb, Vb], axis=1).reshape(2 * HD)
    wall = jnp.concatenate([Qw, kvw], axis=0)
    ball = jnp.concatenate([Qb, kvb])[None, :]

    ew3 = Ew.reshape(H, 2 * D, IN_DIM)
    eww = jnp.concatenate([ew3[:, :D].reshape(HD, IN_DIM),
                           ew3[:, D:].reshape(HD, IN_DIM)], axis=0)
    eb3 = Eb.reshape(H, 2 * D)
    ebw = jnp.concatenate([eb3[:, :D].reshape(HD),
                           eb3[:, D:].reshape(HD)])[None, :]

    eye_h = jnp.eye(H, dtype=f32)
    aw8 = (Aw[:, :, 0].T[:, :, None] * eye_h[:, None, :]).reshape(HD, H)
    awb = jnp.repeat(aw8, D, axis=1)
    vrp = (VeRow.transpose(1, 0, 2)[:, :, None, :] *
           eye_h[:, None, :, None]).reshape(HD, HD)

    z128 = jnp.zeros((NPAD, HD), f32)

    qtab, kv16 = _project(x, wall, ball)
    kvtab = jax.lax.bitcast_convert_type(
        kv16.reshape(N, HD, 2), jnp.int32)

    halves = ((0, H1E), (H1E, H2E))
    gathered = [_gather_edges(kvtab, qtab, src, dst, e0, ne)
                for (e0, ne) in halves]
    batch_wE, accs = None, []
    for (e0, ne), (kvg, qg) in zip(halves, gathered):
        batch_wE, ue_h, uv_h, pexp_h = _edge_compute(
            edge_attr, kvg, qg, eww, ebw, awb, e0, ne, batch_wE)
        accs.append(_scatter_edges(uv_h, ue_h, pexp_h, dst, z128, e0, ne))

    batch_wV = _finalize(accs[0], accs[1], vrp)
    return batch_wV.reshape(N, H, D), batch_wE


# speedup vs baseline: 1.0290x; 1.0074x over previous
import jax
import jax.numpy as jnp
from jax import lax
from jax.experimental import pallas as pl
from jax.experimental.pallas import tpu as pltpu
from jax.experimental.pallas import tpu_sc as plsc

N = 10000
EG = 320000
IN_DIM = 128
D = 16
H = 8
HD = H * D
CLAMP = 5.0

NC = 2
NS = 16
NW = NC * NS

CHUNK = 80
GRP = 4
GEDGES = GRP * CHUNK
SGRP = 2
SEDGES = SGRP * CHUNK

H1E = 163840
H2E = EG - H1E

ROWS_PT = 640
NPAD = ROWS_PT * NS
LAST_ROWS = N - ROWS_PT * (NS - 1)

KB = 2560


def _signed_sqrt(v):
    safe_p = jnp.where(v > 0, v, 1.0)
    safe_n = jnp.where(v < 0, -v, 1.0)
    pos = jnp.where(v > 0, jnp.sqrt(safe_p), 0.0)
    neg = jnp.where(v < 0, jnp.sqrt(safe_n), 0.0)
    return pos - neg


def _k1_body(x_ref, w_ref, b_ref, q_ref, kv_ref):
    tab = jnp.dot(x_ref[...], w_ref[...].T,
                  preferred_element_type=jnp.float32) + b_ref[...]
    q_ref[...] = tab[:, 0:HD]
    kv_ref[...] = tab[:, HD:3 * HD].astype(jnp.bfloat16)


def _project(x, wall, ball):
    return pl.pallas_call(
        _k1_body,
        out_shape=(jax.ShapeDtypeStruct((N, HD), jnp.float32),
                   jax.ShapeDtypeStruct((N, 2 * HD), jnp.bfloat16)),
    )(x, wall, ball)


def _gather_edges(kvtab, qtab, src, dst, e0, ne):
    cpw = ne // NW // CHUNK
    ngrp = cpw // GRP
    leftover = cpw - ngrp * GRP

    def body(kvtab, qtab, src_hbm, dst_hbm, kvg, qg,
             sidx, didx, kvbuf, qbuf, isem, wk, wq):
        cid = lax.axis_index("c")
        sid = lax.axis_index("s")
        wid = sid * NC + cid
        crbase = wid * cpw

        def fire_gathers(nch):
            descs = []
            for j in range(nch):
                descs.append(pltpu.async_copy(
                    kvtab.at[sidx.at[pl.ds(j * CHUNK, CHUNK)]],
                    kvbuf.at[pl.ds(j * CHUNK, CHUNK)], isem))
                descs.append(pltpu.async_copy(
                    qtab.at[didx.at[pl.ds(j * CHUNK, CHUNK)]],
                    qbuf.at[pl.ds(j * CHUNK, CHUNK)], isem))
            for d in descs:
                d.wait()

        def step(t, carry):
            lbase = (crbase + t * GRP) * CHUNK

            @pl.when(t > 0)
            def _():
                pltpu.make_async_copy(kvbuf, kvg.at[pl.ds(lbase, GEDGES)],
                                      wk).wait()
                pltpu.make_async_copy(qbuf, qg.at[pl.ds(lbase, GEDGES)],
                                      wq).wait()

            pltpu.sync_copy(src_hbm.at[pl.ds(e0 + lbase, GEDGES)], sidx)
            pltpu.sync_copy(dst_hbm.at[pl.ds(e0 + lbase, GEDGES)], didx)
            fire_gathers(GRP)
            pltpu.async_copy(kvbuf, kvg.at[pl.ds(lbase, GEDGES)], wk)
            pltpu.async_copy(qbuf, qg.at[pl.ds(lbase, GEDGES)], wq)
            return carry

        lax.fori_loop(0, ngrp, step, 0)

        lbase = (crbase + ngrp * GRP) * CHUNK
        pltpu.make_async_copy(kvbuf, kvg.at[pl.ds(lbase, GEDGES)], wk).wait()
        pltpu.make_async_copy(qbuf, qg.at[pl.ds(lbase, GEDGES)], wq).wait()
        if leftover:
            pltpu.sync_copy(src_hbm.at[pl.ds(e0 + lbase, CHUNK)],
                            sidx.at[pl.ds(0, CHUNK)])
            pltpu.sync_copy(dst_hbm.at[pl.ds(e0 + lbase, CHUNK)],
                            didx.at[pl.ds(0, CHUNK)])
            fire_gathers(1)
            pltpu.sync_copy(kvbuf.at[pl.ds(0, CHUNK)],
                            kvg.at[pl.ds(lbase, CHUNK)])
            pltpu.sync_copy(qbuf.at[pl.ds(0, CHUNK)],
                            qg.at[pl.ds(lbase, CHUNK)])

    mesh = plsc.VectorSubcoreMesh(core_axis_name="c", subcore_axis_name="s",
                                  num_cores=NC, num_subcores=NS)
    kern = pl.kernel(
        body,
        out_type=(jax.ShapeDtypeStruct((ne, HD), jnp.int32),
                  jax.ShapeDtypeStruct((ne, HD), jnp.float32)),
        mesh=mesh,
        scratch_types=[
            pltpu.VMEM((GEDGES,), jnp.int32),
            pltpu.VMEM((GEDGES,), jnp.int32),
            pltpu.VMEM((GEDGES, HD), jnp.int32),
            pltpu.VMEM((GEDGES, HD), jnp.float32),
            pltpu.SemaphoreType.DMA,
            pltpu.SemaphoreType.DMA,
            pltpu.SemaphoreType.DMA,
        ],
    )
    return kern(kvtab, qtab, src, dst)


def _k3_body(ea_ref, kvg_ref, qg_ref, eww_ref, ebw_ref,
             awb_ref, we_ref, ue_ref, uv_ref, pexp_ref):
    ea = ea_ref[...]
    eproj = jnp.dot(ea, eww_ref[...].T,
                    preferred_element_type=jnp.float32) + ebw_ref[...]
    ewp = eproj[:, 0:HD]
    ebp = eproj[:, HD:2 * HD]
    kvi = kvg_ref[...]
    kg = jax.lax.bitcast_convert_type(kvi << 16, jnp.float32)
    vg = jax.lax.bitcast_convert_type(kvi & jnp.int32(-65536), jnp.float32)
    g = kg + qg_ref[...]
    score = _signed_sqrt(g * ewp) + ebp
    we_ref[...] = score
    sb = jnp.dot(score, awb_ref[...], preferred_element_type=jnp.float32)
    pexp = jnp.exp(jnp.clip(sb, -CLAMP, CLAMP))
    pexp_ref[...] = pexp
    ue_ref[...] = score * pexp
    uv_ref[...] = vg * pexp


def _edge_compute(edge_attr, kvg, qg, eww, ebw, awb, e0, ne, we_carry):
    grid = ne // KB
    b0 = e0 // KB
    ea_spec = pl.BlockSpec((KB, HD), lambda i: (i + b0, 0))
    eg_spec = pl.BlockSpec((KB, HD), lambda i: (i, 0))
    full = lambda r, c: pl.BlockSpec((r, c), lambda i: (0, 0))
    eg_out = jax.ShapeDtypeStruct((ne, HD), jnp.float32)
    in_specs = [ea_spec, eg_spec, eg_spec,
                full(2 * HD, IN_DIM), full(1, 2 * HD), full(HD, HD)]
    args = [edge_attr, kvg, qg, eww, ebw, awb]
    aliases = {}
    body = _k3_body
    if we_carry is not None:
        in_specs.append(pl.BlockSpec(memory_space=pl.ANY))
        args.append(we_carry)
        aliases = {6: 0}
        body = lambda ea, kvg_, qg_, eww_, ebw_, awb_, wein, *outs: \
            _k3_body(ea, kvg_, qg_, eww_, ebw_, awb_, *outs)
    return pl.pallas_call(
        body,
        grid=(grid,),
        in_specs=in_specs,
        out_specs=[ea_spec, eg_spec, eg_spec, eg_spec],
        out_shape=[jax.ShapeDtypeStruct((EG, HD), jnp.float32),
                   eg_out, eg_out, eg_out],
        input_output_aliases=aliases,
    )(*args)


def _scatter_edges(uv, ue, pexp, dst, z128, e0, ne):
    n1 = (ne // NS) // CHUNK
    n2 = (ne // 2 // NS) // CHUNK
    pt1 = ne // NS
    pt2 = ne // 2 // NS
    t1 = 0
    t2 = 0

    def body(uv_hbm, ue_hbm, pexp_hbm, dst_hbm, z128,
             accv_out, acce_out, ssa_out, ssb_out,
             sp128, idxa, idxb, bufa, bufb, isem, ssa_sem, ssb_sem):
        cid = lax.axis_index("c")
        sid = lax.axis_index("s")
        rbase = sid * ROWS_PT

        def _zero():
            pltpu.sync_copy(z128.at[pl.ds(rbase, ROWS_PT)],
                            sp128.at[pl.ds(rbase, ROWS_PT)])

        def _load_group(val_hbm, lbase, ixb, dbuf, nch=SGRP):
            descs = []
            for j in range(nch):
                descs.append(pltpu.async_copy(
                    dst_hbm.at[pl.ds(e0 + lbase + j * CHUNK, CHUNK)],
                    ixb.at[j], isem))
            descs.append(pltpu.async_copy(
                val_hbm.at[pl.ds(lbase, nch * CHUNK)],
                dbuf.at[pl.ds(0, nch * CHUNK)], isem))
            for d in descs:
                d.wait()

        def _fire_scatter(ixb, dbuf, sem, nch=SGRP):
            for j in range(nch):
                pltpu.async_copy(dbuf.at[pl.ds(j * CHUNK, CHUNK)],
                                 sp128.at[ixb.at[j]], sem, add=True)

        def _drain_scatter(ixb, dbuf, sem, nch=SGRP):
            for j in range(nch):
                pltpu.make_async_copy(dbuf.at[pl.ds(j * CHUNK, CHUNK)],
                                      sp128.at[ixb.at[j]], sem).wait()

        def _scan(val_hbm, lbase0, n_chunks, tail=0):
            n_pairs = n_chunks // (2 * SGRP)
            rem = n_chunks - 2 * n_pairs * SGRP

            def step(t, carry):
                base_a = lbase0 + (2 * t) * SEDGES
                base_b = base_a + SEDGES

                @pl.when(t > 0)
                def _():
                    _drain_scatter(idxa, bufa, ssa_sem)

                _load_group(val_hbm, base_a, idxa, bufa)
                _fire_scatter(idxa, bufa, ssa_sem)

                @pl.when(t > 0)
                def _():
                    _drain_scatter(idxb, bufb, ssb_sem)

                _load_group(val_hbm, base_b, idxb, bufb)
                _fire_scatter(idxb, bufb, ssb_sem)
                return carry
            lax.fori_loop(0, n_pairs, step, 0)
            _drain_scatter(idxa, bufa, ssa_sem)
            base = lbase0 + 2 * n_pairs * SEDGES
            while rem > 0:
                nch = min(SGRP, rem)
                _load_group(val_hbm, base, idxa, bufa, nch)
                _fire_scatter(idxa, bufa, ssa_sem, nch)
                _drain_scatter(idxa, bufa, ssa_sem, nch)
                base += nch * CHUNK
                rem -= nch
            _drain_scatter(idxb, bufb, ssb_sem)

        def _writeback(out_hbm):
            @pl.when(sid < NS - 1)
            def _():
                pltpu.sync_copy(sp128.at[pl.ds(rbase, ROWS_PT)],
                                out_hbm.at[pl.ds(rbase, ROWS_PT)])

            @pl.when(sid == NS - 1)
            def _():
                pltpu.sync_copy(sp128.at[pl.ds(rbase, LAST_ROWS)],
                                out_hbm.at[pl.ds(rbase, LAST_ROWS)])

        _zero()
        plsc.subcore_barrier()

        @pl.when(cid == 0)
        def _():
            _scan(uv_hbm, sid * pt1, n1, t1)

        @pl.when(cid == 1)
        def _():
            _scan(ue_hbm, sid * pt1, n1, t1)

        plsc.subcore_barrier()

        @pl.when(cid == 0)
        def _():
            _writeback(accv_out)

        @pl.when(cid == 1)
        def _():
            _writeback(acce_out)

        plsc.subcore_barrier()

        _zero()
        plsc.subcore_barrier()
        _scan(pexp_hbm, cid * (ne // 2) + sid * pt2, n2, t2)
        plsc.subcore_barrier()

        @pl.when(cid == 0)
        def _():
            _writeback(ssa_out)

        @pl.when(cid == 1)
        def _():
            _writeback(ssb_out)

    mesh = plsc.VectorSubcoreMesh(core_axis_name="c", subcore_axis_name="s",
                                  num_cores=NC, num_subcores=NS)
    n_out = jax.ShapeDtypeStruct((N, HD), jnp.float32)
    kern = pl.kernel(
        body,
        out_type=(n_out, n_out, n_out, n_out),
        mesh=mesh,
        scratch_types=[
            pltpu.VMEM_SHARED((NPAD, HD), jnp.float32),
            pltpu.VMEM((SGRP, CHUNK), jnp.int32),
            pltpu.VMEM((SGRP, CHUNK), jnp.int32),
            pltpu.VMEM((SEDGES, HD), jnp.float32),
            pltpu.VMEM((SEDGES, HD), jnp.float32),
            pltpu.SemaphoreType.DMA,
            pltpu.SemaphoreType.DMA,
            pltpu.SemaphoreType.DMA,
        ],
    )
    return kern(uv, ue, pexp, dst, z128)


def _k5_body(av1, av2, ae1, ae2, sa1, sb1, sa2, sb2, vrp_ref, out_ref):
    ssum = sa1[...] + sb1[...] + sa2[...] + sb2[...]
    rexp = 1.0 / (ssum + 1e-16)
    acce = ae1[...] + ae2[...]
    rowv = jnp.dot(acce * rexp, vrp_ref[...],
                   preferred_element_type=jnp.float32)
    out_ref[...] = (av1[...] + av2[...]) * rexp + rowv


def _finalize(accs1, accs2, vrp):
    av1, ae1, sa1, sb1 = accs1
    av2, ae2, sa2, sb2 = accs2
    return pl.pallas_call(
        _k5_body,
        out_shape=jax.ShapeDtypeStruct((N, HD), jnp.float32),
    )(av1, av2, ae1, ae2, sa1, sb1, sa2, sb2, vrp)


def kernel(x, edge_attr, edge_index, Qw, Qb, Kw, Kb, Ew, Eb, Vw, Vb, Aw, VeRow):
    f32 = jnp.float32
    x = x.astype(f32)
    edge_attr = edge_attr.astype(f32)
    src = edge_index[0].astype(jnp.int32)
    dst = edge_index[1].astype(jnp.int32)

    kvw = jnp.stack([Kw, Vw], axis=1).reshape(2 * HD, IN_DIM)
    kvb = jnp.stack([Kb, Vb], axis=1).reshape(2 * HD)
    wall = jnp.concatenate([Qw, kvw], axis=0)
    ball = jnp.concatenate([Qb, kvb])[None, :]

    ew3 = Ew.reshape(H, 2 * D, IN_DIM)
    eww = jnp.concatenate([ew3[:, :D].reshape(HD, IN_DIM),
                           ew3[:, D:].reshape(HD, IN_DIM)], axis=0)
    eb3 = Eb.reshape(H, 2 * D)
    ebw = jnp.concatenate([eb3[:, :D].reshape(HD),
                           eb3[:, D:].reshape(HD)])[None, :]

    eye_h = jnp.eye(H, dtype=f32)
    aw8 = (Aw[:, :, 0].T[:, :, None] * eye_h[:, None, :]).reshape(HD, H)
    awb = jnp.repeat(aw8, D, axis=1)
    vrp = (VeRow.transpose(1, 0, 2)[:, :, None, :] *
           eye_h[:, None, :, None]).reshape(HD, HD)

    z128 = jnp.zeros((NPAD, HD), f32)

    qtab, kv16 = _project(x, wall, ball)
    kvtab = jax.lax.bitcast_convert_type(
        kv16.reshape(N, HD, 2), jnp.int32)

    halves = ((0, H1E), (H1E, H2E))
    gathered = [_gather_edges(kvtab, qtab, src, dst, e0, ne)
                for (e0, ne) in halves]
    batch_wE, accs = None, []
    for (e0, ne), (kvg, qg) in zip(halves, gathered):
        batch_wE, ue_h, uv_h, pexp_h = _edge_compute(
            edge_attr, kvg, qg, eww, ebw, awb, e0, ne, batch_wE)
        accs.append(_scatter_edges(uv_h, ue_h, pexp_h, dst, z128, e0, ne))

    batch_wV = _finalize(accs[0], accs[1], vrp)
    return batch_wV.reshape(N, H, D), batch_wE


# speedup vs baseline: 1.0294x; 1.0004x over previous
import jax
import jax.numpy as jnp
from jax import lax
from jax.experimental import pallas as pl
from jax.experimental.pallas import tpu as pltpu
from jax.experimental.pallas import tpu_sc as plsc

N = 10000
EG = 320000
IN_DIM = 128
D = 16
H = 8
HD = H * D
CLAMP = 5.0

NC = 2
NS = 16
NW = NC * NS

CHUNK = 80
GRP = 4
GEDGES = GRP * CHUNK
SGRP = 2
SEDGES = SGRP * CHUNK

H1E = 163840
H2E = EG - H1E

ROWS_PT = 640
NPAD = ROWS_PT * NS
LAST_ROWS = N - ROWS_PT * (NS - 1)

KB = 2560


def _signed_sqrt(v):
    safe_p = jnp.where(v > 0, v, 1.0)
    safe_n = jnp.where(v < 0, -v, 1.0)
    pos = jnp.where(v > 0, jnp.sqrt(safe_p), 0.0)
    neg = jnp.where(v < 0, jnp.sqrt(safe_n), 0.0)
    return pos - neg


def _k1_body(x_ref, w_ref, b_ref, q_ref, kv_ref):
    tab = jnp.dot(x_ref[...], w_ref[...].T,
                  preferred_element_type=jnp.float32) + b_ref[...]
    q_ref[...] = tab[:, 0:HD]
    kv_ref[...] = tab[:, HD:3 * HD].astype(jnp.bfloat16)


def _project(x, wall, ball):
    return pl.pallas_call(
        _k1_body,
        out_shape=(jax.ShapeDtypeStruct((N, HD), jnp.float32),
                   jax.ShapeDtypeStruct((N, 2 * HD), jnp.bfloat16)),
    )(x, wall, ball)


def _gather_edges(kvtab, qtab, src, dst, e0, ne):
    cpw = ne // NW // CHUNK
    ngrp = cpw // GRP
    leftover = cpw - ngrp * GRP

    def body(kvtab, qtab, src_hbm, dst_hbm, kvg, qg,
             sidx, didx, kvbuf, qbuf, isem, wk, wq):
        cid = lax.axis_index("c")
        sid = lax.axis_index("s")
        wid = sid * NC + cid
        crbase = wid * cpw

        def fire_gathers(nch):
            descs = []
            for j in range(nch):
                descs.append(pltpu.async_copy(
                    kvtab.at[sidx.at[pl.ds(j * CHUNK, CHUNK)]],
                    kvbuf.at[pl.ds(j * CHUNK, CHUNK)], isem))
                descs.append(pltpu.async_copy(
                    qtab.at[didx.at[pl.ds(j * CHUNK, CHUNK)]],
                    qbuf.at[pl.ds(j * CHUNK, CHUNK)], isem))
            for d in descs:
                d.wait()

        def step(t, carry):
            lbase = (crbase + t * GRP) * CHUNK

            @pl.when(t > 0)
            def _():
                pltpu.make_async_copy(kvbuf, kvg.at[pl.ds(lbase, GEDGES)],
                                      wk).wait()
                pltpu.make_async_copy(qbuf, qg.at[pl.ds(lbase, GEDGES)],
                                      wq).wait()

            pltpu.sync_copy(src_hbm.at[pl.ds(e0 + lbase, GEDGES)], sidx)
            pltpu.sync_copy(dst_hbm.at[pl.ds(e0 + lbase, GEDGES)], didx)
            fire_gathers(GRP)
            pltpu.async_copy(kvbuf, kvg.at[pl.ds(lbase, GEDGES)], wk)
            pltpu.async_copy(qbuf, qg.at[pl.ds(lbase, GEDGES)], wq)
            return carry

        lax.fori_loop(0, ngrp, step, 0)

        lbase = (crbase + ngrp * GRP) * CHUNK
        pltpu.make_async_copy(kvbuf, kvg.at[pl.ds(lbase, GEDGES)], wk).wait()
        pltpu.make_async_copy(qbuf, qg.at[pl.ds(lbase, GEDGES)], wq).wait()
        if leftover:
            lsz = leftover * CHUNK
            pltpu.sync_copy(src_hbm.at[pl.ds(e0 + lbase, lsz)],
                            sidx.at[pl.ds(0, lsz)])
            pltpu.sync_copy(dst_hbm.at[pl.ds(e0 + lbase, lsz)],
                            didx.at[pl.ds(0, lsz)])
            fire_gathers(leftover)
            pltpu.sync_copy(kvbuf.at[pl.ds(0, lsz)],
                            kvg.at[pl.ds(lbase, lsz)])
            pltpu.sync_copy(qbuf.at[pl.ds(0, lsz)],
                            qg.at[pl.ds(lbase, lsz)])

    mesh = plsc.VectorSubcoreMesh(core_axis_name="c", subcore_axis_name="s",
                                  num_cores=NC, num_subcores=NS)
    kern = pl.kernel(
        body,
        out_type=(jax.ShapeDtypeStruct((ne, HD), jnp.int32),
                  jax.ShapeDtypeStruct((ne, HD), jnp.float32)),
        mesh=mesh,
        scratch_types=[
            pltpu.VMEM((GEDGES,), jnp.int32),
            pltpu.VMEM((GEDGES,), jnp.int32),
            pltpu.VMEM((GEDGES, HD), jnp.int32),
            pltpu.VMEM((GEDGES, HD), jnp.float32),
            pltpu.SemaphoreType.DMA,
            pltpu.SemaphoreType.DMA,
            pltpu.SemaphoreType.DMA,
        ],
    )
    return kern(kvtab, qtab, src, dst)


def _k3_body(ea_ref, kvg_ref, qg_ref, eww_ref, ebw_ref,
             awb_ref, we_ref, ue_ref, uv_ref, pexp_ref):
    ea = ea_ref[...]
    eproj = jnp.dot(ea, eww_ref[...].T,
                    preferred_element_type=jnp.float32) + ebw_ref[...]
    ewp = eproj[:, 0:HD]
    ebp = eproj[:, HD:2 * HD]
    kvi = kvg_ref[...]
    kg = jax.lax.bitcast_convert_type(kvi << 16, jnp.float32)
    vg = jax.lax.bitcast_convert_type(kvi & jnp.int32(-65536), jnp.float32)
    g = kg + qg_ref[...]
    score = _signed_sqrt(g * ewp) + ebp
    we_ref[...] = score
    sb = jnp.dot(score, awb_ref[...], preferred_element_type=jnp.float32)
    pexp = jnp.exp(jnp.clip(sb, -CLAMP, CLAMP))
    pexp_ref[...] = pexp
    ue_ref[...] = score * pexp
    uv_ref[...] = vg * pexp


def _edge_compute(edge_attr, kvg, qg, eww, ebw, awb, e0, ne, we_carry):
    grid = ne // KB
    b0 = e0 // KB
    ea_spec = pl.BlockSpec((KB, HD), lambda i: (i + b0, 0))
    eg_spec = pl.BlockSpec((KB, HD), lambda i: (i, 0))
    full = lambda r, c: pl.BlockSpec((r, c), lambda i: (0, 0))
    eg_out = jax.ShapeDtypeStruct((ne, HD), jnp.float32)
    in_specs = [ea_spec, eg_spec, eg_spec,
                full(2 * HD, IN_DIM), full(1, 2 * HD), full(HD, HD)]
    args = [edge_attr, kvg, qg, eww, ebw, awb]
    aliases = {}
    body = _k3_body
    if we_carry is not None:
        in_specs.append(pl.BlockSpec(memory_space=pl.ANY))
        args.append(we_carry)
        aliases = {6: 0}
        body = lambda ea, kvg_, qg_, eww_, ebw_, awb_, wein, *outs: \
            _k3_body(ea, kvg_, qg_, eww_, ebw_, awb_, *outs)
    return pl.pallas_call(
        body,
        grid=(grid,),
        in_specs=in_specs,
        out_specs=[ea_spec, eg_spec, eg_spec, eg_spec],
        out_shape=[jax.ShapeDtypeStruct((EG, HD), jnp.float32),
                   eg_out, eg_out, eg_out],
        input_output_aliases=aliases,
    )(*args)


def _scatter_edges(uv, ue, pexp, dst, z128, e0, ne):
    n1 = (ne // NS) // CHUNK
    n2 = (ne // 2 // NS) // CHUNK
    pt1 = ne // NS
    pt2 = ne // 2 // NS
    t1 = 0
    t2 = 0

    def body(uv_hbm, ue_hbm, pexp_hbm, dst_hbm, z128,
             accv_out, acce_out, ssa_out, ssb_out,
             sp128, idxa, idxb, bufa, bufb, isem, ssa_sem, ssb_sem):
        cid = lax.axis_index("c")
        sid = lax.axis_index("s")
        rbase = sid * ROWS_PT

        def _zero():
            pltpu.sync_copy(z128.at[pl.ds(rbase, ROWS_PT)],
                            sp128.at[pl.ds(rbase, ROWS_PT)])

        def _load_group(val_hbm, lbase, ixb, dbuf, nch=SGRP):
            descs = []
            for j in range(nch):
                descs.append(pltpu.async_copy(
                    dst_hbm.at[pl.ds(e0 + lbase + j * CHUNK, CHUNK)],
                    ixb.at[j], isem))
            descs.append(pltpu.async_copy(
                val_hbm.at[pl.ds(lbase, nch * CHUNK)],
                dbuf.at[pl.ds(0, nch * CHUNK)], isem))
            for d in descs:
                d.wait()

        def _fire_scatter(ixb, dbuf, sem, nch=SGRP):
            for j in range(nch):
                pltpu.async_copy(dbuf.at[pl.ds(j * CHUNK, CHUNK)],
                                 sp128.at[ixb.at[j]], sem, add=True)

        def _drain_scatter(ixb, dbuf, sem, nch=SGRP):
            for j in range(nch):
                pltpu.make_async_copy(dbuf.at[pl.ds(j * CHUNK, CHUNK)],
                                      sp128.at[ixb.at[j]], sem).wait()

        def _scan(val_hbm, lbase0, n_chunks, tail=0):
            n_pairs = n_chunks // (2 * SGRP)
            rem = n_chunks - 2 * n_pairs * SGRP

            def step(t, carry):
                base_a = lbase0 + (2 * t) * SEDGES
                base_b = base_a + SEDGES

                @pl.when(t > 0)
                def _():
                    _drain_scatter(idxa, bufa, ssa_sem)

                _load_group(val_hbm, base_a, idxa, bufa)
                _fire_scatter(idxa, bufa, ssa_sem)

                @pl.when(t > 0)
                def _():
                    _drain_scatter(idxb, bufb, ssb_sem)

                _load_group(val_hbm, base_b, idxb, bufb)
                _fire_scatter(idxb, bufb, ssb_sem)
                return carry
            lax.fori_loop(0, n_pairs, step, 0)
            _drain_scatter(idxa, bufa, ssa_sem)
            base = lbase0 + 2 * n_pairs * SEDGES
            while rem > 0:
                nch = min(SGRP, rem)
                _load_group(val_hbm, base, idxa, bufa, nch)
                _fire_scatter(idxa, bufa, ssa_sem, nch)
                _drain_scatter(idxa, bufa, ssa_sem, nch)
                base += nch * CHUNK
                rem -= nch
            _drain_scatter(idxb, bufb, ssb_sem)

        def _writeback(out_hbm):
            @pl.when(sid < NS - 1)
            def _():
                pltpu.sync_copy(sp128.at[pl.ds(rbase, ROWS_PT)],
                                out_hbm.at[pl.ds(rbase, ROWS_PT)])

            @pl.when(sid == NS - 1)
            def _():
                pltpu.sync_copy(sp128.at[pl.ds(rbase, LAST_ROWS)],
                                out_hbm.at[pl.ds(rbase, LAST_ROWS)])

        _zero()
        plsc.subcore_barrier()

        @pl.when(cid == 0)
        def _():
            _scan(uv_hbm, sid * pt1, n1, t1)

        @pl.when(cid == 1)
        def _():
            _scan(ue_hbm, sid * pt1, n1, t1)

        plsc.subcore_barrier()

        @pl.when(cid == 0)
        def _():
            _writeback(accv_out)

        @pl.when(cid == 1)
        def _():
            _writeback(acce_out)

        plsc.subcore_barrier()

        _zero()
        plsc.subcore_barrier()
        _scan(pexp_hbm, cid * (ne // 2) + sid * pt2, n2, t2)
        plsc.subcore_barrier()

        @pl.when(cid == 0)
        def _():
            _writeback(ssa_out)

        @pl.when(cid == 1)
        def _():
            _writeback(ssb_out)

    mesh = plsc.VectorSubcoreMesh(core_axis_name="c", subcore_axis_name="s",
                                  num_cores=NC, num_subcores=NS)
    n_out = jax.ShapeDtypeStruct((N, HD), jnp.float32)
    kern = pl.kernel(
        body,
        out_type=(n_out, n_out, n_out, n_out),
        mesh=mesh,
        scratch_types=[
            pltpu.VMEM_SHARED((NPAD, HD), jnp.float32),
            pltpu.VMEM((SGRP, CHUNK), jnp.int32),
            pltpu.VMEM((SGRP, CHUNK), jnp.int32),
            pltpu.VMEM((SEDGES, HD), jnp.float32),
            pltpu.VMEM((SEDGES, HD), jnp.float32),
            pltpu.SemaphoreType.DMA,
            pltpu.SemaphoreType.DMA,
            pltpu.SemaphoreType.DMA,
        ],
    )
    return kern(uv, ue, pexp, dst, z128)


def _k5_body(av1, av2, ae1, ae2, sa1, sb1, sa2, sb2, vrp_ref, out_ref):
    ssum = sa1[...] + sb1[...] + sa2[...] + sb2[...]
    rexp = 1.0 / (ssum + 1e-16)
    acce = ae1[...] + ae2[...]
    rowv = jnp.dot(acce * rexp, vrp_ref[...],
                   preferred_element_type=jnp.float32)
    out_ref[...] = (av1[...] + av2[...]) * rexp + rowv


def _finalize(accs1, accs2, vrp):
    av1, ae1, sa1, sb1 = accs1
    av2, ae2, sa2, sb2 = accs2
    return pl.pallas_call(
        _k5_body,
        out_shape=jax.ShapeDtypeStruct((N, HD), jnp.float32),
    )(av1, av2, ae1, ae2, sa1, sb1, sa2, sb2, vrp)


def kernel(x, edge_attr, edge_index, Qw, Qb, Kw, Kb, Ew, Eb, Vw, Vb, Aw, VeRow):
    f32 = jnp.float32
    x = x.astype(f32)
    edge_attr = edge_attr.astype(f32)
    src = edge_index[0].astype(jnp.int32)
    dst = edge_index[1].astype(jnp.int32)

    kvw = jnp.stack([Kw, Vw], axis=1).reshape(2 * HD, IN_DIM)
    kvb = jnp.stack([Kb, Vb], axis=1).reshape(2 * HD)
    wall = jnp.concatenate([Qw, kvw], axis=0)
    ball = jnp.concatenate([Qb, kvb])[None, :]

    ew3 = Ew.reshape(H, 2 * D, IN_DIM)
    eww = jnp.concatenate([ew3[:, :D].reshape(HD, IN_DIM),
                           ew3[:, D:].reshape(HD, IN_DIM)], axis=0)
    eb3 = Eb.reshape(H, 2 * D)
    ebw = jnp.concatenate([eb3[:, :D].reshape(HD),
                           eb3[:, D:].reshape(HD)])[None, :]

    eye_h = jnp.eye(H, dtype=f32)
    aw8 = (Aw[:, :, 0].T[:, :, None] * eye_h[:, None, :]).reshape(HD, H)
    awb = jnp.repeat(aw8, D, axis=1)
    vrp = (VeRow.transpose(1, 0, 2)[:, :, None, :] *
           eye_h[:, None, :, None]).reshape(HD, HD)

    z128 = jnp.zeros((NPAD, HD), f32)

    qtab, kv16 = _project(x, wall, ball)
    kvtab = jax.lax.bitcast_convert_type(
        kv16.reshape(N, HD, 2), jnp.int32)

    halves = ((0, H1E), (H1E, H2E))
    gathered = [_gather_edges(kvtab, qtab, src, dst, e0, ne)
                for (e0, ne) in halves]
    batch_wE, accs = None, []
    for (e0, ne), (kvg, qg) in zip(halves, gathered):
        batch_wE, ue_h, uv_h, pexp_h = _edge_compute(
            edge_attr, kvg, qg, eww, ebw, awb, e0, ne, batch_wE)
        accs.append(_scatter_edges(uv_h, ue_h, pexp_h, dst, z128, e0, ne))

    batch_wV = _finalize(accs[0], accs[1], vrp)
    return batch_wV.reshape(N, H, D), batch_wE
